# baseline (speedup 1.0000x reference)
import jax
import jax.numpy as jnp
from jax.experimental import pallas as pl

HIDDEN_DIM = 2048
NUM_CHOICES = 64
BLOCK_M = 512


def _gate_kernel(x_ref, w_ref, b_ref, o_ref):
    logits = jax.lax.dot_general(
        x_ref[...], w_ref[...],
        dimension_numbers=(((1,), (1,)), ((), ())),
        preferred_element_type=jnp.float32,
    )
    logits = logits + b_ref[...]
    row_max = jnp.max(logits, axis=1, keepdims=True)
    col = jax.lax.broadcasted_iota(jnp.int32, logits.shape, 1)
    cand = jnp.where(logits == row_max, col, NUM_CHOICES)
    idx = jnp.min(cand, axis=1, keepdims=True)
    o_ref[...] = (col == idx).astype(o_ref.dtype)


def kernel(x, W, b):
    n = x.shape[0]
    b2 = b.reshape(1, NUM_CHOICES)
    return pl.pallas_call(
        _gate_kernel,
        grid=(n // BLOCK_M,),
        in_specs=[
            pl.BlockSpec((BLOCK_M, HIDDEN_DIM), lambda i: (i, 0)),
            pl.BlockSpec((NUM_CHOICES, HIDDEN_DIM), lambda i: (0, 0)),
            pl.BlockSpec((1, NUM_CHOICES), lambda i: (0, 0)),
        ],
        out_specs=pl.BlockSpec((BLOCK_M, NUM_CHOICES), lambda i: (i, 0)),
        out_shape=jax.ShapeDtypeStruct((n, NUM_CHOICES), x.dtype),
    )(x, W, b2)
